# baseline (speedup 1.0000x reference)
import functools

import jax
import jax.numpy as jnp
from jax import lax
from jax.experimental import pallas as pl
from jax.experimental.pallas import tpu as pltpu
from jax.experimental.pallas import tpu_sc as plsc

H = 128
NB = 8
BM = 1000
CH = 128
NW = 32


def _sc_gather_body(per_w, table_hbm, idx_hbm, out_hbm,
                    iv0, iv1, rv0, rv1, gs0, gs1, os0, os1):
    wid = lax.axis_index("s") * 2 + lax.axis_index("c")

    def chunk(i, iv, rv, gs, osm):
        base = (wid + i * NW) * CH
        @pl.when(i >= 2)
        def _():
            pltpu.make_async_copy(rv, out_hbm.at[pl.ds(0, CH)], osm).wait()

        pltpu.sync_copy(idx_hbm.at[pl.ds(base, CH)], iv)
        pltpu.async_copy(table_hbm.at[iv], rv, gs).wait()
        pltpu.async_copy(rv, out_hbm.at[pl.ds(base, CH)], osm)

    def body(k, carry):
        chunk(2 * k, iv0, rv0, gs0, os0)
        chunk(2 * k + 1, iv1, rv1, gs1, os1)
        return carry

    lax.fori_loop(0, per_w // 2, body, 0)
    if per_w % 2:
        chunk(per_w - 1, iv0, rv0, gs0, os0)
    pltpu.make_async_copy(rv0, out_hbm.at[pl.ds(0, CH)], os0).wait()
    if per_w >= 2:
        pltpu.make_async_copy(rv1, out_hbm.at[pl.ds(0, CH)], os1).wait()


def _sc_gather(table, idx):
    B = idx.shape[0]
    pad = (-B) % (CH * NW)
    if pad:
        idx = jnp.pad(idx, (0, pad))
    n_chunks = (B + pad) // CH
    per_w = n_chunks // NW
    W = table.shape[1]
    mesh = plsc.VectorSubcoreMesh(core_axis_name="c", subcore_axis_name="s")
    run = functools.partial(
        pl.kernel,
        mesh=mesh,
        compiler_params=pltpu.CompilerParams(use_tc_tiling_on_sc=False),
        out_type=jax.ShapeDtypeStruct((B + pad, W), table.dtype),
        scratch_types=[
            pltpu.VMEM((CH,), jnp.int32),
            pltpu.VMEM((CH,), jnp.int32),
            pltpu.VMEM((CH, W), table.dtype),
            pltpu.VMEM((CH, W), table.dtype),
            pltpu.SemaphoreType.DMA,
            pltpu.SemaphoreType.DMA,
            pltpu.SemaphoreType.DMA,
            pltpu.SemaphoreType.DMA,
        ],
    )(functools.partial(_sc_gather_body, per_w))
    out = run(table, idx)
    return out[:B] if pad else out


def _sig(v):
    return 1.0 / (1.0 + jnp.exp(-v))


def _pack_bf16(h):
    hb = h.astype(jnp.bfloat16).astype(jnp.float32)
    t = lax.bitcast_convert_type(hb, jnp.uint32)
    return (t[:, :H // 2] >> 16) | (t[:, H // 2:] & jnp.uint32(0xFFFF0000))


def _unpack_bf16(u):
    lo = lax.bitcast_convert_type(u << 16, jnp.float32)
    hi = lax.bitcast_convert_type(u & jnp.uint32(0xFFFF0000), jnp.float32)
    return jnp.concatenate([lo, hi], axis=-1)


def _first_step_body(x_ref, wzt_ref, bz_ref, wht_ref, bh_ref, h_ref):
    x = x_ref[...]
    z = _sig(jnp.dot(x, wzt_ref[...], preferred_element_type=jnp.float32)
             + bz_ref[...])
    p = jnp.tanh(jnp.dot(x, wht_ref[...], preferred_element_type=jnp.float32)
                 + bh_ref[...])
    h_ref[...] = _pack_bf16(z * p)


def _first_step(x, Wz_w, Wz_b, Wh_w, Wh_b):
    E = x.shape[0]
    grid = (E // BM,)
    return pl.pallas_call(
        _first_step_body,
        grid=grid,
        in_specs=[
            pl.BlockSpec((BM, H), lambda i: (i, 0)),
            pl.BlockSpec((H, H), lambda i: (0, 0)),
            pl.BlockSpec((1, H), lambda i: (0, 0)),
            pl.BlockSpec((H, H), lambda i: (0, 0)),
            pl.BlockSpec((1, H), lambda i: (0, 0)),
        ],
        out_specs=pl.BlockSpec((BM, H // 2), lambda i: (i, 0)),
        out_shape=jax.ShapeDtypeStruct((E, H // 2), jnp.uint32),
    )(x, Wz_w[:H], Wz_b.reshape(1, H), Wh_w[:H], Wh_b.reshape(1, H))


def _gru_body(final, x_ref, hnei_ref, wz_ref, bz_ref, wr_ref, ur_ref, bu_ref,
              wh_ref, bh_ref, h_ref):
    x = x_ref[...]
    hnei = _unpack_bf16(hnei_ref[...])
    sum_h = jnp.sum(hnei, axis=1)
    z = _sig(jnp.dot(x, wz_ref[0:H], preferred_element_type=jnp.float32)
             + jnp.dot(sum_h, wz_ref[H:2 * H],
                       preferred_element_type=jnp.float32)
             + bz_ref[...])
    r1 = jnp.dot(x, wr_ref[...], preferred_element_type=jnp.float32)
    r2 = lax.dot_general(hnei, ur_ref[...], (((2,), (0,)), ((), ())),
                         preferred_element_type=jnp.float32)
    r = _sig(r1[:, None, :] + r2 + bu_ref[...][None])
    sum_g = jnp.sum(r * hnei, axis=1)
    p = jnp.tanh(jnp.dot(x, wh_ref[0:H], preferred_element_type=jnp.float32)
                 + jnp.dot(sum_g, wh_ref[H:2 * H],
                           preferred_element_type=jnp.float32)
                 + bh_ref[...])
    h = (1.0 - z) * sum_h + z * p
    h_ref[...] = h if final else _pack_bf16(h)


def _gru_step(x, hnei, Wz_w, Wz_b, Wr_w, Ur_w, Ur_b, Wh_w, Wh_b,
              final=False):
    E = x.shape[0]
    grid = (E // BM,)
    out_w = H if final else H // 2
    out_dt = jnp.float32 if final else jnp.uint32
    return pl.pallas_call(
        functools.partial(_gru_body, final),
        grid=grid,
        in_specs=[
            pl.BlockSpec((BM, H), lambda i: (i, 0)),
            pl.BlockSpec((BM, NB, H // 2), lambda i: (i, 0, 0)),
            pl.BlockSpec((2 * H, H), lambda i: (0, 0)),
            pl.BlockSpec((1, H), lambda i: (0, 0)),
            pl.BlockSpec((H, H), lambda i: (0, 0)),
            pl.BlockSpec((H, H), lambda i: (0, 0)),
            pl.BlockSpec((1, H), lambda i: (0, 0)),
            pl.BlockSpec((2 * H, H), lambda i: (0, 0)),
            pl.BlockSpec((1, H), lambda i: (0, 0)),
        ],
        out_specs=pl.BlockSpec((BM, out_w), lambda i: (i, 0)),
        out_shape=jax.ShapeDtypeStruct((E, out_w), out_dt),
    )(x, hnei, Wz_w, Wz_b.reshape(1, H), Wr_w, Ur_w, Ur_b.reshape(1, H),
      Wh_w, Wh_b.reshape(1, H))


def _root_body(xr_ref, hrnei_ref, w_ref, b_ref, out_ref):
    xr = xr_ref[...]
    s = jnp.sum(hrnei_ref[...], axis=1)
    v = (jnp.dot(xr, w_ref[0:H], preferred_element_type=jnp.float32)
         + jnp.dot(s, w_ref[H:2 * H], preferred_element_type=jnp.float32)
         + b_ref[...])
    out_ref[...] = jnp.maximum(v, 0.0)


def _root_agg(x_root, hrnei, W_w, W_b):
    R = x_root.shape[0]
    return pl.pallas_call(
        _root_body,
        grid=(1,),
        in_specs=[
            pl.BlockSpec((R, H), lambda i: (0, 0)),
            pl.BlockSpec((R, NB, H), lambda i: (0, 0, 0)),
            pl.BlockSpec((2 * H, H), lambda i: (0, 0)),
            pl.BlockSpec((1, H), lambda i: (0, 0)),
        ],
        out_specs=pl.BlockSpec((R, H), lambda i: (0, 0)),
        out_shape=jax.ShapeDtypeStruct((R, H), jnp.float32),
    )(x_root, hrnei, W_w, W_b.reshape(1, H))


def kernel(fmess_wid, mess_nei, root_wid, root_mess_nei, embedding,
           Wz_w, Wz_b, Wr_w, Ur_w, Ur_b, Wh_w, Wh_b, W_w, W_b):
    E = fmess_wid.shape[0]
    T = 6

    x = _sc_gather(embedding, fmess_wid)[:E]
    hpk = _first_step(x, Wz_w, Wz_b, Wh_w, Wh_b)
    flat_nei = mess_nei.reshape(-1)
    for t in range(T - 1):
        final = t == T - 2
        hnei = _sc_gather(hpk, flat_nei).reshape(E, NB, H // 2)
        out = _gru_step(x, hnei, Wz_w, Wz_b, Wr_w, Ur_w, Ur_b, Wh_w, Wh_b,
                        final=final)
        if final:
            h = out
        else:
            hpk = out

    R = root_wid.shape[0]
    x_root = _sc_gather(embedding, root_wid)[:R]
    hrnei = _sc_gather(h, root_mess_nei.reshape(-1)).reshape(R, NB, H)
    root_vecs = _root_agg(x_root, hrnei, W_w, W_b)
    return (h, root_vecs)


# speedup vs baseline: 1.2061x; 1.2061x over previous
import functools

import jax
import jax.numpy as jnp
from jax import lax
from jax.experimental import pallas as pl
from jax.experimental.pallas import tpu as pltpu
from jax.experimental.pallas import tpu_sc as plsc

H = 128
NB = 8
BM = 1000
CH = 128
NW = 32


def _sc_gather_body(per_w, table_hbm, idx_hbm, out_hbm,
                    iv0, iv1, rv0, rv1, gs0, gs1, os0, os1):
    wid = lax.axis_index("s") * 2 + lax.axis_index("c")

    def chunk(i, iv, rv, gs, osm):
        base = (wid + i * NW) * CH
        @pl.when(i >= 2)
        def _():
            pltpu.make_async_copy(rv, out_hbm.at[pl.ds(0, CH)], osm).wait()

        pltpu.sync_copy(idx_hbm.at[pl.ds(base, CH)], iv)
        pltpu.async_copy(table_hbm.at[iv], rv, gs).wait()
        pltpu.async_copy(rv, out_hbm.at[pl.ds(base, CH)], osm)

    def body(k, carry):
        chunk(2 * k, iv0, rv0, gs0, os0)
        chunk(2 * k + 1, iv1, rv1, gs1, os1)
        return carry

    lax.fori_loop(0, per_w // 2, body, 0)
    if per_w % 2:
        chunk(per_w - 1, iv0, rv0, gs0, os0)
    pltpu.make_async_copy(rv0, out_hbm.at[pl.ds(0, CH)], os0).wait()
    if per_w >= 2:
        pltpu.make_async_copy(rv1, out_hbm.at[pl.ds(0, CH)], os1).wait()


def _sc_gather(table, idx):
    B = idx.shape[0]
    pad = (-B) % (CH * NW)
    if pad:
        idx = jnp.pad(idx, (0, pad))
    n_chunks = (B + pad) // CH
    per_w = n_chunks // NW
    W = table.shape[1]
    mesh = plsc.VectorSubcoreMesh(core_axis_name="c", subcore_axis_name="s")
    run = functools.partial(
        pl.kernel,
        mesh=mesh,
        compiler_params=pltpu.CompilerParams(use_tc_tiling_on_sc=False),
        out_type=jax.ShapeDtypeStruct((B + pad, W), table.dtype),
        scratch_types=[
            pltpu.VMEM((CH,), jnp.int32),
            pltpu.VMEM((CH,), jnp.int32),
            pltpu.VMEM((CH, W), table.dtype),
            pltpu.VMEM((CH, W), table.dtype),
            pltpu.SemaphoreType.DMA,
            pltpu.SemaphoreType.DMA,
            pltpu.SemaphoreType.DMA,
            pltpu.SemaphoreType.DMA,
        ],
    )(functools.partial(_sc_gather_body, per_w))
    return run(table, idx)


def _sig(v):
    return 1.0 / (1.0 + jnp.exp(-v))


def _pack_bf16(h):
    hb = h.astype(jnp.bfloat16).astype(jnp.float32)
    t = lax.bitcast_convert_type(hb, jnp.uint32)
    return (t[:, :H // 2] >> 16) | (t[:, H // 2:] & jnp.uint32(0xFFFF0000))


def _unpack_bf16(u):
    lo = lax.bitcast_convert_type(u << 16, jnp.float32)
    hi = lax.bitcast_convert_type(u & jnp.uint32(0xFFFF0000), jnp.float32)
    return jnp.concatenate([lo, hi], axis=-1)


def _first_step_body(x_ref, wzt_ref, bz_ref, wht_ref, bh_ref, h_ref):
    x = x_ref[...]
    z = _sig(jnp.dot(x, wzt_ref[...], preferred_element_type=jnp.float32)
             + bz_ref[...])
    p = jnp.tanh(jnp.dot(x, wht_ref[...], preferred_element_type=jnp.float32)
                 + bh_ref[...])
    h_ref[...] = _pack_bf16(z * p)


def _first_step(E, x, Wz_w, Wz_b, Wh_w, Wh_b):
    grid = (E // BM,)
    return pl.pallas_call(
        _first_step_body,
        grid=grid,
        in_specs=[
            pl.BlockSpec((BM, H), lambda i: (i, 0)),
            pl.BlockSpec((H, H), lambda i: (0, 0)),
            pl.BlockSpec((1, H), lambda i: (0, 0)),
            pl.BlockSpec((H, H), lambda i: (0, 0)),
            pl.BlockSpec((1, H), lambda i: (0, 0)),
        ],
        out_specs=pl.BlockSpec((BM, H // 2), lambda i: (i, 0)),
        out_shape=jax.ShapeDtypeStruct((E, H // 2), jnp.uint32),
    )(x, Wz_w[:H], Wz_b.reshape(1, H), Wh_w[:H], Wh_b.reshape(1, H))


def _gru_body(final, x_ref, hnei_ref, wz_ref, bz_ref, wr_ref, ur_ref, bu_ref,
              wh_ref, bh_ref, h_ref):
    x = x_ref[...]
    hnei = _unpack_bf16(hnei_ref[...])
    sum_h = jnp.sum(hnei, axis=1)
    z = _sig(jnp.dot(x, wz_ref[0:H], preferred_element_type=jnp.float32)
             + jnp.dot(sum_h, wz_ref[H:2 * H],
                       preferred_element_type=jnp.float32)
             + bz_ref[...])
    r1 = jnp.dot(x, wr_ref[...], preferred_element_type=jnp.float32)
    r2 = lax.dot_general(hnei, ur_ref[...], (((2,), (0,)), ((), ())),
                         preferred_element_type=jnp.float32)
    r = _sig(r1[:, None, :] + r2 + bu_ref[...][None])
    sum_g = jnp.sum(r * hnei, axis=1)
    p = jnp.tanh(jnp.dot(x, wh_ref[0:H], preferred_element_type=jnp.float32)
                 + jnp.dot(sum_g, wh_ref[H:2 * H],
                           preferred_element_type=jnp.float32)
                 + bh_ref[...])
    h = (1.0 - z) * sum_h + z * p
    h_ref[...] = h if final else _pack_bf16(h)


def _gru_step(E, x, hnei, Wz_w, Wz_b, Wr_w, Ur_w, Ur_b, Wh_w, Wh_b,
              final=False):
    grid = (E // BM,)
    out_w = H if final else H // 2
    out_dt = jnp.float32 if final else jnp.uint32
    return pl.pallas_call(
        functools.partial(_gru_body, final),
        grid=grid,
        in_specs=[
            pl.BlockSpec((BM, H), lambda i: (i, 0)),
            pl.BlockSpec((BM, NB, H // 2), lambda i: (i, 0, 0)),
            pl.BlockSpec((2 * H, H), lambda i: (0, 0)),
            pl.BlockSpec((1, H), lambda i: (0, 0)),
            pl.BlockSpec((H, H), lambda i: (0, 0)),
            pl.BlockSpec((H, H), lambda i: (0, 0)),
            pl.BlockSpec((1, H), lambda i: (0, 0)),
            pl.BlockSpec((2 * H, H), lambda i: (0, 0)),
            pl.BlockSpec((1, H), lambda i: (0, 0)),
        ],
        out_specs=pl.BlockSpec((BM, out_w), lambda i: (i, 0)),
        out_shape=jax.ShapeDtypeStruct((E, out_w), out_dt),
    )(x, hnei, Wz_w, Wz_b.reshape(1, H), Wr_w, Ur_w, Ur_b.reshape(1, H),
      Wh_w, Wh_b.reshape(1, H))


def _root_body(xr_ref, hrnei_ref, w_ref, b_ref, out_ref):
    xr = xr_ref[...]
    s = jnp.sum(hrnei_ref[...], axis=1)
    v = (jnp.dot(xr, w_ref[0:H], preferred_element_type=jnp.float32)
         + jnp.dot(s, w_ref[H:2 * H], preferred_element_type=jnp.float32)
         + b_ref[...])
    out_ref[...] = jnp.maximum(v, 0.0)


def _root_agg(R, x_root, hrnei, W_w, W_b):
    return pl.pallas_call(
        _root_body,
        grid=(1,),
        in_specs=[
            pl.BlockSpec((R, H), lambda i: (0, 0)),
            pl.BlockSpec((R, NB, H), lambda i: (0, 0, 0)),
            pl.BlockSpec((2 * H, H), lambda i: (0, 0)),
            pl.BlockSpec((1, H), lambda i: (0, 0)),
        ],
        out_specs=pl.BlockSpec((R, H), lambda i: (0, 0)),
        out_shape=jax.ShapeDtypeStruct((R, H), jnp.float32),
    )(x_root, hrnei, W_w, W_b.reshape(1, H))


def kernel(fmess_wid, mess_nei, root_wid, root_mess_nei, embedding,
           Wz_w, Wz_b, Wr_w, Ur_w, Ur_b, Wh_w, Wh_b, W_w, W_b):
    E = fmess_wid.shape[0]
    T = 6

    x = _sc_gather(embedding, fmess_wid)
    hpk = _first_step(E, x, Wz_w, Wz_b, Wh_w, Wh_b)
    flat_nei = mess_nei.reshape(-1)
    for t in range(T - 1):
        final = t == T - 2
        hnei = _sc_gather(hpk, flat_nei).reshape(-1, NB, H // 2)
        out = _gru_step(E, x, hnei, Wz_w, Wz_b, Wr_w, Ur_w, Ur_b, Wh_w, Wh_b,
                        final=final)
        if final:
            h = out
        else:
            hpk = out

    R = root_wid.shape[0]
    x_root = _sc_gather(embedding, root_wid)
    hrnei = _sc_gather(h, root_mess_nei.reshape(-1)).reshape(-1, NB, H)
    root_vecs = _root_agg(R, x_root, hrnei, W_w, W_b)
    return (h, root_vecs)


# speedup vs baseline: 1.5668x; 1.2991x over previous
import functools

import jax
import jax.numpy as jnp
from jax import lax
from jax.experimental import pallas as pl
from jax.experimental.pallas import tpu as pltpu
from jax.experimental.pallas import tpu_sc as plsc

H = 128
NB = 8
BM = 1000
CH = 128
NW = 32


def _sc_gather_body(per_w, table_hbm, idx_hbm, out_hbm,
                    iv0, iv1, rv0, rv1, gs0, gs1, os0, os1):
    wid = lax.axis_index("s") * 2 + lax.axis_index("c")

    def chunk(i, iv, rv, gs, osm):
        base = (wid + i * NW) * CH
        @pl.when(i >= 2)
        def _():
            pltpu.make_async_copy(rv, out_hbm.at[pl.ds(0, CH)], osm).wait()

        pltpu.sync_copy(idx_hbm.at[pl.ds(base, CH)], iv)
        pltpu.async_copy(table_hbm.at[iv], rv, gs).wait()
        pltpu.async_copy(rv, out_hbm.at[pl.ds(base, CH)], osm)

    def body(k, carry):
        chunk(2 * k, iv0, rv0, gs0, os0)
        chunk(2 * k + 1, iv1, rv1, gs1, os1)
        return carry

    lax.fori_loop(0, per_w // 2, body, 0)
    if per_w % 2:
        chunk(per_w - 1, iv0, rv0, gs0, os0)
    pltpu.make_async_copy(rv0, out_hbm.at[pl.ds(0, CH)], os0).wait()
    if per_w >= 2:
        pltpu.make_async_copy(rv1, out_hbm.at[pl.ds(0, CH)], os1).wait()


def _sc_gather(table, idx):
    B = idx.shape[0]
    pad = (-B) % (CH * NW)
    if pad:
        idx = jnp.pad(idx, (0, pad))
    n_chunks = (B + pad) // CH
    per_w = n_chunks // NW
    W = table.shape[1]
    mesh = plsc.VectorSubcoreMesh(core_axis_name="c", subcore_axis_name="s")
    run = functools.partial(
        pl.kernel,
        mesh=mesh,
        compiler_params=pltpu.CompilerParams(use_tc_tiling_on_sc=False),
        out_type=jax.ShapeDtypeStruct((B + pad, W), table.dtype),
        scratch_types=[
            pltpu.VMEM((CH,), jnp.int32),
            pltpu.VMEM((CH,), jnp.int32),
            pltpu.VMEM((CH, W), table.dtype),
            pltpu.VMEM((CH, W), table.dtype),
            pltpu.SemaphoreType.DMA,
            pltpu.SemaphoreType.DMA,
            pltpu.SemaphoreType.DMA,
            pltpu.SemaphoreType.DMA,
        ],
    )(functools.partial(_sc_gather_body, per_w))
    return run(table, idx)


def _sig(v):
    return 1.0 / (1.0 + jnp.exp(-v))


def _pack_bf16(h):
    hb = h.astype(jnp.bfloat16).astype(jnp.float32)
    t = lax.bitcast_convert_type(hb, jnp.uint32)
    return (t[:, :H // 2] >> 16) | (t[:, H // 2:] & jnp.uint32(0xFFFF0000))


def _unpack_bf16(u):
    lo = lax.bitcast_convert_type(u << 16, jnp.float32)
    hi = lax.bitcast_convert_type(u & jnp.uint32(0xFFFF0000), jnp.float32)
    return jnp.concatenate([lo, hi], axis=-1)


def _first_step_body(x_ref, wzt_ref, bz_ref, wht_ref, bh_ref, h_ref):
    x = x_ref[...]
    z = _sig(jnp.dot(x, wzt_ref[...], preferred_element_type=jnp.float32)
             + bz_ref[...])
    p = jnp.tanh(jnp.dot(x, wht_ref[...], preferred_element_type=jnp.float32)
                 + bh_ref[...])
    h_ref[...] = _pack_bf16(z * p)


def _first_step(E, x, Wz_w, Wz_b, Wh_w, Wh_b):
    grid = (E // BM,)
    return pl.pallas_call(
        _first_step_body,
        grid=grid,
        in_specs=[
            pl.BlockSpec((BM, H), lambda i: (i, 0)),
            pl.BlockSpec((H, H), lambda i: (0, 0)),
            pl.BlockSpec((1, H), lambda i: (0, 0)),
            pl.BlockSpec((H, H), lambda i: (0, 0)),
            pl.BlockSpec((1, H), lambda i: (0, 0)),
        ],
        out_specs=pl.BlockSpec((BM, H // 2), lambda i: (i, 0)),
        out_shape=jax.ShapeDtypeStruct((E, H // 2), jnp.uint32),
    )(x, Wz_w[:H], Wz_b.reshape(1, H), Wh_w[:H], Wh_b.reshape(1, H))


def _gru_body(final, x_ref, hnei_ref, wz_ref, bz_ref, wr_ref, ur_ref, bu_ref,
              wh_ref, bh_ref, h_ref):
    x = x_ref[...]
    u = hnei_ref[...]
    lo = lax.bitcast_convert_type(u << 16, jnp.float32)
    hi = lax.bitcast_convert_type(u & jnp.uint32(0xFFFF0000), jnp.float32)
    hw = H // 2
    ev = jnp.concatenate([lo[:, :hw], hi[:, :hw]], axis=-1)
    od = jnp.concatenate([lo[:, hw:], hi[:, hw:]], axis=-1)
    hnei = jnp.concatenate([ev.reshape(BM, NB // 2, H),
                            od.reshape(BM, NB // 2, H)], axis=1)
    sum_h = jnp.sum(hnei, axis=1)
    z = _sig(jnp.dot(x, wz_ref[0:H], preferred_element_type=jnp.float32)
             + jnp.dot(sum_h, wz_ref[H:2 * H],
                       preferred_element_type=jnp.float32)
             + bz_ref[...])
    r1 = jnp.dot(x, wr_ref[...], preferred_element_type=jnp.float32)
    r2 = lax.dot_general(hnei, ur_ref[...], (((2,), (0,)), ((), ())),
                         preferred_element_type=jnp.float32)
    r = _sig(r1[:, None, :] + r2 + bu_ref[...][None])
    sum_g = jnp.sum(r * hnei, axis=1)
    p = jnp.tanh(jnp.dot(x, wh_ref[0:H], preferred_element_type=jnp.float32)
                 + jnp.dot(sum_g, wh_ref[H:2 * H],
                           preferred_element_type=jnp.float32)
                 + bh_ref[...])
    h = (1.0 - z) * sum_h + z * p
    h_ref[...] = h if final else _pack_bf16(h)


def _gru_step(E, x, hnei, Wz_w, Wz_b, Wr_w, Ur_w, Ur_b, Wh_w, Wh_b,
              final=False):
    grid = (E // BM,)
    out_w = H if final else H // 2
    out_dt = jnp.float32 if final else jnp.uint32
    return pl.pallas_call(
        functools.partial(_gru_body, final),
        grid=grid,
        in_specs=[
            pl.BlockSpec((BM, H), lambda i: (i, 0)),
            pl.BlockSpec((BM * NB // 2, H), lambda i: (i, 0)),
            pl.BlockSpec((2 * H, H), lambda i: (0, 0)),
            pl.BlockSpec((1, H), lambda i: (0, 0)),
            pl.BlockSpec((H, H), lambda i: (0, 0)),
            pl.BlockSpec((H, H), lambda i: (0, 0)),
            pl.BlockSpec((1, H), lambda i: (0, 0)),
            pl.BlockSpec((2 * H, H), lambda i: (0, 0)),
            pl.BlockSpec((1, H), lambda i: (0, 0)),
        ],
        out_specs=pl.BlockSpec((BM, out_w), lambda i: (i, 0)),
        out_shape=jax.ShapeDtypeStruct((E, out_w), out_dt),
    )(x, hnei, Wz_w, Wz_b.reshape(1, H), Wr_w, Ur_w, Ur_b.reshape(1, H),
      Wh_w, Wh_b.reshape(1, H))


def _root_body(xr_ref, hrnei_ref, w_ref, b_ref, out_ref):
    xr = xr_ref[...]
    s = jnp.sum(hrnei_ref[...], axis=1)
    v = (jnp.dot(xr, w_ref[0:H], preferred_element_type=jnp.float32)
         + jnp.dot(s, w_ref[H:2 * H], preferred_element_type=jnp.float32)
         + b_ref[...])
    out_ref[...] = jnp.maximum(v, 0.0)


def _root_agg(R, x_root, hrnei, W_w, W_b):
    return pl.pallas_call(
        _root_body,
        grid=(1,),
        in_specs=[
            pl.BlockSpec((R, H), lambda i: (0, 0)),
            pl.BlockSpec((R, NB, H), lambda i: (0, 0, 0)),
            pl.BlockSpec((2 * H, H), lambda i: (0, 0)),
            pl.BlockSpec((1, H), lambda i: (0, 0)),
        ],
        out_specs=pl.BlockSpec((R, H), lambda i: (0, 0)),
        out_shape=jax.ShapeDtypeStruct((R, H), jnp.float32),
    )(x_root, hrnei, W_w, W_b.reshape(1, H))


def kernel(fmess_wid, mess_nei, root_wid, root_mess_nei, embedding,
           Wz_w, Wz_b, Wr_w, Ur_w, Ur_b, Wh_w, Wh_b, W_w, W_b):
    E = fmess_wid.shape[0]
    T = 6

    x = _sc_gather(embedding, fmess_wid)
    hpk = _first_step(E, x, Wz_w, Wz_b, Wh_w, Wh_b)
    flat_nei = mess_nei.reshape(-1)
    for t in range(T - 1):
        final = t == T - 2
        hnei = _sc_gather(hpk, flat_nei).reshape(-1, H)
        out = _gru_step(E, x, hnei, Wz_w, Wz_b, Wr_w, Ur_w, Ur_b, Wh_w, Wh_b,
                        final=final)
        if final:
            h = out
        else:
            hpk = out

    R = root_wid.shape[0]
    x_root = _sc_gather(embedding, root_wid)
    hrnei = _sc_gather(h, root_mess_nei.reshape(-1)).reshape(-1, NB, H)
    root_vecs = _root_agg(R, x_root, hrnei, W_w, W_b)
    return (h, root_vecs)


# speedup vs baseline: 1.8742x; 1.1962x over previous
import functools

import jax
import jax.numpy as jnp
from jax import lax
from jax.experimental import pallas as pl
from jax.experimental.pallas import tpu as pltpu
from jax.experimental.pallas import tpu_sc as plsc

H = 128
NB = 8
BM = 1000
CH = 128
NW = 32


def _sc_gather_body(per_w, table_hbm, idx_hbm, out_hbm,
                    iv0, iv1, rv0, rv1, gs0, gs1, os0, os1):
    wid = lax.axis_index("s") * 2 + lax.axis_index("c")

    def chunk(i, iv, rv, gs, osm):
        base = (wid + i * NW) * CH
        @pl.when(i >= 2)
        def _():
            pltpu.make_async_copy(rv, out_hbm.at[pl.ds(0, CH)], osm).wait()

        pltpu.sync_copy(idx_hbm.at[pl.ds(base, CH)], iv)
        pltpu.async_copy(table_hbm.at[iv], rv, gs).wait()
        pltpu.async_copy(rv, out_hbm.at[pl.ds(base, CH)], osm)

    def body(k, carry):
        chunk(2 * k, iv0, rv0, gs0, os0)
        chunk(2 * k + 1, iv1, rv1, gs1, os1)
        return carry

    lax.fori_loop(0, per_w // 2, body, 0)
    if per_w % 2:
        chunk(per_w - 1, iv0, rv0, gs0, os0)
    pltpu.make_async_copy(rv0, out_hbm.at[pl.ds(0, CH)], os0).wait()
    if per_w >= 2:
        pltpu.make_async_copy(rv1, out_hbm.at[pl.ds(0, CH)], os1).wait()


def _sc_gather(table, idx):
    B = idx.shape[0]
    pad = (-B) % (CH * NW)
    if pad:
        idx = jnp.pad(idx, (0, pad))
    n_chunks = (B + pad) // CH
    per_w = n_chunks // NW
    W = table.shape[1]
    mesh = plsc.VectorSubcoreMesh(core_axis_name="c", subcore_axis_name="s")
    run = functools.partial(
        pl.kernel,
        mesh=mesh,
        compiler_params=pltpu.CompilerParams(use_tc_tiling_on_sc=False),
        out_type=jax.ShapeDtypeStruct((B + pad, W), table.dtype),
        scratch_types=[
            pltpu.VMEM((CH,), jnp.int32),
            pltpu.VMEM((CH,), jnp.int32),
            pltpu.VMEM((CH, W), table.dtype),
            pltpu.VMEM((CH, W), table.dtype),
            pltpu.SemaphoreType.DMA,
            pltpu.SemaphoreType.DMA,
            pltpu.SemaphoreType.DMA,
            pltpu.SemaphoreType.DMA,
        ],
    )(functools.partial(_sc_gather_body, per_w))
    return run(table, idx)


def _sig(v):
    return 0.5 + 0.5 * jnp.tanh(0.5 * v)


def _pack_bf16(h):
    hb = h.astype(jnp.bfloat16).astype(jnp.float32)
    t = lax.bitcast_convert_type(hb, jnp.uint32)
    return (t[:, :H // 2] >> 16) | (t[:, H // 2:] & jnp.uint32(0xFFFF0000))


def _unpack_bf16(u):
    lo = lax.bitcast_convert_type(u << 16, jnp.float32)
    hi = lax.bitcast_convert_type(u & jnp.uint32(0xFFFF0000), jnp.float32)
    return jnp.concatenate([lo, hi], axis=-1)


def _first_step_body(x_ref, wzt_ref, bz_ref, wht_ref, bh_ref, h_ref):
    x = x_ref[...]
    z = _sig(jnp.dot(x, wzt_ref[...], preferred_element_type=jnp.float32)
             + bz_ref[...])
    p = jnp.tanh(jnp.dot(x, wht_ref[...], preferred_element_type=jnp.float32)
                 + bh_ref[...])
    h_ref[...] = _pack_bf16(z * p)


def _first_step(E, x, Wz_w, Wz_b, Wh_w, Wh_b):
    grid = (E // BM,)
    return pl.pallas_call(
        _first_step_body,
        grid=grid,
        in_specs=[
            pl.BlockSpec((BM, H), lambda i: (i, 0)),
            pl.BlockSpec((H, H), lambda i: (0, 0)),
            pl.BlockSpec((1, H), lambda i: (0, 0)),
            pl.BlockSpec((H, H), lambda i: (0, 0)),
            pl.BlockSpec((1, H), lambda i: (0, 0)),
        ],
        out_specs=pl.BlockSpec((BM, H // 2), lambda i: (i, 0)),
        out_shape=jax.ShapeDtypeStruct((E, H // 2), jnp.uint32),
    )(x, Wz_w[:H], Wz_b.reshape(1, H), Wh_w[:H], Wh_b.reshape(1, H))


def _gru_body(final, x_ref, hnei_ref, wz_ref, bz_ref, wr_ref, ur_ref, bu_ref,
              wh_ref, bh_ref, h_ref):
    x = x_ref[...]
    u = hnei_ref[...]
    lo = lax.bitcast_convert_type(u << 16, jnp.float32)
    hi = lax.bitcast_convert_type(u & jnp.uint32(0xFFFF0000), jnp.float32)
    hw = H // 2
    ev = jnp.concatenate([lo[:, :hw], hi[:, :hw]], axis=-1)
    od = jnp.concatenate([lo[:, hw:], hi[:, hw:]], axis=-1)
    hnei = jnp.concatenate([ev.reshape(BM, NB // 2, H),
                            od.reshape(BM, NB // 2, H)], axis=1)
    sum_h = jnp.sum(hnei, axis=1)
    z = _sig(jnp.dot(x, wz_ref[0:H], preferred_element_type=jnp.float32)
             + jnp.dot(sum_h, wz_ref[H:2 * H],
                       preferred_element_type=jnp.float32)
             + bz_ref[...])
    r1 = jnp.dot(x, wr_ref[...], preferred_element_type=jnp.float32) \
        + bu_ref[...]
    r2 = lax.dot_general(hnei, ur_ref[...], (((2,), (0,)), ((), ())),
                         preferred_element_type=jnp.float32)
    r = _sig(r1[:, None, :] + r2)
    sum_g = jnp.sum(r * hnei, axis=1)
    p = jnp.tanh(jnp.dot(x, wh_ref[0:H], preferred_element_type=jnp.float32)
                 + jnp.dot(sum_g, wh_ref[H:2 * H],
                           preferred_element_type=jnp.float32)
                 + bh_ref[...])
    h = (1.0 - z) * sum_h + z * p
    h_ref[...] = h if final else _pack_bf16(h)


def _gru_step(E, x, hnei, Wz_w, Wz_b, Wr_w, Ur_w, Ur_b, Wh_w, Wh_b,
              final=False, x_off=0):
    grid = (E // BM,)
    out_w = H if final else H // 2
    out_dt = jnp.float32 if final else jnp.uint32
    return pl.pallas_call(
        functools.partial(_gru_body, final),
        grid=grid,
        in_specs=[
            pl.BlockSpec((BM, H), lambda i: (i + x_off, 0)),
            pl.BlockSpec((BM * NB // 2, H), lambda i: (i, 0)),
            pl.BlockSpec((2 * H, H), lambda i: (0, 0)),
            pl.BlockSpec((1, H), lambda i: (0, 0)),
            pl.BlockSpec((H, H), lambda i: (0, 0)),
            pl.BlockSpec((H, H), lambda i: (0, 0)),
            pl.BlockSpec((1, H), lambda i: (0, 0)),
            pl.BlockSpec((2 * H, H), lambda i: (0, 0)),
            pl.BlockSpec((1, H), lambda i: (0, 0)),
        ],
        out_specs=pl.BlockSpec((BM, out_w), lambda i: (i, 0)),
        out_shape=jax.ShapeDtypeStruct((E, out_w), out_dt),
    )(x, hnei, Wz_w, Wz_b.reshape(1, H), Wr_w, Ur_w, Ur_b.reshape(1, H),
      Wh_w, Wh_b.reshape(1, H))


def _root_body(xr_ref, hrnei_ref, w_ref, b_ref, out_ref):
    xr = xr_ref[...]
    s = jnp.sum(hrnei_ref[...], axis=1)
    v = (jnp.dot(xr, w_ref[0:H], preferred_element_type=jnp.float32)
         + jnp.dot(s, w_ref[H:2 * H], preferred_element_type=jnp.float32)
         + b_ref[...])
    out_ref[...] = jnp.maximum(v, 0.0)


def _root_agg(R, x_root, hrnei, W_w, W_b):
    return pl.pallas_call(
        _root_body,
        grid=(1,),
        in_specs=[
            pl.BlockSpec((R, H), lambda i: (0, 0)),
            pl.BlockSpec((R, NB, H), lambda i: (0, 0, 0)),
            pl.BlockSpec((2 * H, H), lambda i: (0, 0)),
            pl.BlockSpec((1, H), lambda i: (0, 0)),
        ],
        out_specs=pl.BlockSpec((R, H), lambda i: (0, 0)),
        out_shape=jax.ShapeDtypeStruct((R, H), jnp.float32),
    )(x_root, hrnei, W_w, W_b.reshape(1, H))


def kernel(fmess_wid, mess_nei, root_wid, root_mess_nei, embedding,
           Wz_w, Wz_b, Wr_w, Ur_w, Ur_b, Wh_w, Wh_b, W_w, W_b):
    E = fmess_wid.shape[0]
    T = 6

    x = _sc_gather(embedding, fmess_wid)
    hpk = _first_step(E, x, Wz_w, Wz_b, Wh_w, Wh_b)
    EH = E // 2
    flat_nei = mess_nei.reshape(-1)
    nei_a, nei_b = flat_nei[:EH * NB], flat_nei[EH * NB:]
    gru = functools.partial(_gru_step, EH, x, Wz_w=Wz_w, Wz_b=Wz_b,
                            Wr_w=Wr_w, Ur_w=Ur_w, Ur_b=Ur_b, Wh_w=Wh_w,
                            Wh_b=Wh_b)
    for t in range(T - 1):
        final = t == T - 2
        ga = _sc_gather(hpk, nei_a)
        gb = _sc_gather(hpk, nei_b)
        oa = gru(hnei=ga.reshape(-1, H), final=final, x_off=0)
        ob = gru(hnei=gb.reshape(-1, H), final=final, x_off=EH // BM)
        out = jnp.concatenate([oa, ob], axis=0)
        if final:
            h = out
        else:
            hpk = out

    R = root_wid.shape[0]
    x_root = _sc_gather(embedding, root_wid)
    hrnei = _sc_gather(h, root_mess_nei.reshape(-1)).reshape(-1, NB, H)
    root_vecs = _root_agg(R, x_root, hrnei, W_w, W_b)
    return (h, root_vecs)


# speedup vs baseline: 1.9841x; 1.0586x over previous
import functools

import jax
import jax.numpy as jnp
from jax import lax
from jax.experimental import pallas as pl
from jax.experimental.pallas import tpu as pltpu
from jax.experimental.pallas import tpu_sc as plsc

H = 128
NB = 8
BM = 1000
CH = 128
NW = 32


def _sc_gather_body(per_w, table_hbm, idx_hbm, out_hbm,
                    iv0, iv1, rv0, rv1, gs0, gs1, os0, os1):
    wid = lax.axis_index("s") * 2 + lax.axis_index("c")

    def chunk(i, iv, rv, gs, osm):
        base = (wid + i * NW) * CH
        @pl.when(i >= 2)
        def _():
            pltpu.make_async_copy(rv, out_hbm.at[pl.ds(0, CH)], osm).wait()

        pltpu.sync_copy(idx_hbm.at[pl.ds(base, CH)], iv)
        pltpu.async_copy(table_hbm.at[iv], rv, gs).wait()
        pltpu.async_copy(rv, out_hbm.at[pl.ds(base, CH)], osm)

    def body(k, carry):
        chunk(2 * k, iv0, rv0, gs0, os0)
        chunk(2 * k + 1, iv1, rv1, gs1, os1)
        return carry

    lax.fori_loop(0, per_w // 2, body, 0)
    if per_w % 2:
        chunk(per_w - 1, iv0, rv0, gs0, os0)
    pltpu.make_async_copy(rv0, out_hbm.at[pl.ds(0, CH)], os0).wait()
    if per_w >= 2:
        pltpu.make_async_copy(rv1, out_hbm.at[pl.ds(0, CH)], os1).wait()


def _sc_gather(table, idx):
    B = idx.shape[0]
    pad = (-B) % (CH * NW)
    if pad:
        idx = jnp.pad(idx, (0, pad))
    n_chunks = (B + pad) // CH
    per_w = n_chunks // NW
    W = table.shape[1]
    mesh = plsc.VectorSubcoreMesh(core_axis_name="c", subcore_axis_name="s")
    run = functools.partial(
        pl.kernel,
        mesh=mesh,
        compiler_params=pltpu.CompilerParams(use_tc_tiling_on_sc=False),
        out_type=jax.ShapeDtypeStruct((B + pad, W), table.dtype),
        scratch_types=[
            pltpu.VMEM((CH,), jnp.int32),
            pltpu.VMEM((CH,), jnp.int32),
            pltpu.VMEM((CH, W), table.dtype),
            pltpu.VMEM((CH, W), table.dtype),
            pltpu.SemaphoreType.DMA,
            pltpu.SemaphoreType.DMA,
            pltpu.SemaphoreType.DMA,
            pltpu.SemaphoreType.DMA,
        ],
    )(functools.partial(_sc_gather_body, per_w))
    return run(table, idx)


def _sig(v):
    return 0.5 + 0.5 * jnp.tanh(0.5 * v)


def _pack_bf16(h):
    hb = h.astype(jnp.bfloat16).astype(jnp.float32)
    t = lax.bitcast_convert_type(hb, jnp.uint32)
    return (t[:, :H // 2] >> 16) | (t[:, H // 2:] & jnp.uint32(0xFFFF0000))


def _unpack_bf16(u):
    lo = lax.bitcast_convert_type(u << 16, jnp.float32)
    hi = lax.bitcast_convert_type(u & jnp.uint32(0xFFFF0000), jnp.float32)
    return jnp.concatenate([lo, hi], axis=-1)


def _first_step_body(x_ref, wzt_ref, bz_ref, wht_ref, bh_ref, h_ref):
    x = x_ref[...]
    z = _sig(jnp.dot(x, wzt_ref[...], preferred_element_type=jnp.float32)
             + bz_ref[...])
    p = jnp.tanh(jnp.dot(x, wht_ref[...], preferred_element_type=jnp.float32)
                 + bh_ref[...])
    h_ref[...] = _pack_bf16(z * p)


def _first_step(E, x, Wz_w, Wz_b, Wh_w, Wh_b):
    grid = (E // BM,)
    return pl.pallas_call(
        _first_step_body,
        grid=grid,
        in_specs=[
            pl.BlockSpec((BM, H), lambda i: (i, 0)),
            pl.BlockSpec((H, H), lambda i: (0, 0)),
            pl.BlockSpec((1, H), lambda i: (0, 0)),
            pl.BlockSpec((H, H), lambda i: (0, 0)),
            pl.BlockSpec((1, H), lambda i: (0, 0)),
        ],
        out_specs=pl.BlockSpec((BM, H // 2), lambda i: (i, 0)),
        out_shape=jax.ShapeDtypeStruct((E, H // 2), jnp.uint32),
    )(x, Wz_w[:H], Wz_b.reshape(1, H), Wh_w[:H], Wh_b.reshape(1, H))


def _gru_body(final, x_ref, hnei_ref, wz_ref, bz_ref, wr_ref, ur_ref, bu_ref,
              wh_ref, bh_ref, h_ref):
    x = x_ref[...]
    u = hnei_ref[...]
    lo = lax.bitcast_convert_type(u << 16, jnp.float32)
    hi = lax.bitcast_convert_type(u & jnp.uint32(0xFFFF0000), jnp.float32)
    hw = H // 2
    ev = jnp.concatenate([lo[:, :hw], hi[:, :hw]], axis=-1)
    od = jnp.concatenate([lo[:, hw:], hi[:, hw:]], axis=-1)
    hnei = jnp.concatenate([ev.reshape(BM, NB // 2, H),
                            od.reshape(BM, NB // 2, H)], axis=1)
    sum_h = jnp.sum(hnei, axis=1)
    z = _sig(jnp.dot(x, wz_ref[0:H], preferred_element_type=jnp.float32)
             + jnp.dot(sum_h, wz_ref[H:2 * H],
                       preferred_element_type=jnp.float32)
             + bz_ref[...])
    r1 = jnp.dot(x, wr_ref[...], preferred_element_type=jnp.float32) \
        + bu_ref[...]
    r2 = lax.dot_general(hnei, ur_ref[...], (((2,), (0,)), ((), ())),
                         preferred_element_type=jnp.float32)
    r = _sig(r1[:, None, :] + r2)
    sum_g = jnp.sum(r * hnei, axis=1)
    p = jnp.tanh(jnp.dot(x, wh_ref[0:H], preferred_element_type=jnp.float32)
                 + jnp.dot(sum_g, wh_ref[H:2 * H],
                           preferred_element_type=jnp.float32)
                 + bh_ref[...])
    h = (1.0 - z) * sum_h + z * p
    h_ref[...] = h if final else _pack_bf16(h)


def _gru_step(E, x, hnei, Wz_w, Wz_b, Wr_w, Ur_w, Ur_b, Wh_w, Wh_b,
              final=False, x_off=0):
    grid = (E // BM,)
    out_w = H if final else H // 2
    out_dt = jnp.float32 if final else jnp.uint32
    return pl.pallas_call(
        functools.partial(_gru_body, final),
        grid=grid,
        in_specs=[
            pl.BlockSpec((BM, H), lambda i: (i + x_off, 0)),
            pl.BlockSpec((BM * NB // 2, H), lambda i: (i, 0)),
            pl.BlockSpec((2 * H, H), lambda i: (0, 0)),
            pl.BlockSpec((1, H), lambda i: (0, 0)),
            pl.BlockSpec((H, H), lambda i: (0, 0)),
            pl.BlockSpec((H, H), lambda i: (0, 0)),
            pl.BlockSpec((1, H), lambda i: (0, 0)),
            pl.BlockSpec((2 * H, H), lambda i: (0, 0)),
            pl.BlockSpec((1, H), lambda i: (0, 0)),
        ],
        out_specs=pl.BlockSpec((BM, out_w), lambda i: (i, 0)),
        out_shape=jax.ShapeDtypeStruct((E, out_w), out_dt),
    )(x, hnei, Wz_w, Wz_b.reshape(1, H), Wr_w, Ur_w, Ur_b.reshape(1, H),
      Wh_w, Wh_b.reshape(1, H))


def _root_body(xr_ref, hrnei_ref, w_ref, b_ref, out_ref):
    xr = xr_ref[...]
    s = jnp.sum(hrnei_ref[...], axis=1)
    v = (jnp.dot(xr, w_ref[0:H], preferred_element_type=jnp.float32)
         + jnp.dot(s, w_ref[H:2 * H], preferred_element_type=jnp.float32)
         + b_ref[...])
    out_ref[...] = jnp.maximum(v, 0.0)


def _root_agg(R, x_root, hrnei, W_w, W_b):
    return pl.pallas_call(
        _root_body,
        grid=(1,),
        in_specs=[
            pl.BlockSpec((R, H), lambda i: (0, 0)),
            pl.BlockSpec((R, NB, H), lambda i: (0, 0, 0)),
            pl.BlockSpec((2 * H, H), lambda i: (0, 0)),
            pl.BlockSpec((1, H), lambda i: (0, 0)),
        ],
        out_specs=pl.BlockSpec((R, H), lambda i: (0, 0)),
        out_shape=jax.ShapeDtypeStruct((R, H), jnp.float32),
    )(x_root, hrnei, W_w, W_b.reshape(1, H))


def kernel(fmess_wid, mess_nei, root_wid, root_mess_nei, embedding,
           Wz_w, Wz_b, Wr_w, Ur_w, Ur_b, Wh_w, Wh_b, W_w, W_b):
    E = fmess_wid.shape[0]
    T = 6

    x = _sc_gather(embedding, fmess_wid)
    hpk = _first_step(E, x, Wz_w, Wz_b, Wh_w, Wh_b)
    NSPLIT = 4
    EH = E // NSPLIT
    flat_nei = mess_nei.reshape(-1)
    neis = [flat_nei[c * EH * NB:(c + 1) * EH * NB] for c in range(NSPLIT)]
    gru = functools.partial(_gru_step, EH, x, Wz_w=Wz_w, Wz_b=Wz_b,
                            Wr_w=Wr_w, Ur_w=Ur_w, Ur_b=Ur_b, Wh_w=Wh_w,
                            Wh_b=Wh_b)
    for t in range(T - 1):
        final = t == T - 2
        gs = [_sc_gather(hpk, n) for n in neis]
        os_ = [gru(hnei=g.reshape(-1, H), final=final,
                   x_off=c * (EH // BM)) for c, g in enumerate(gs)]
        out = jnp.concatenate(os_, axis=0)
        if final:
            h = out
        else:
            hpk = out

    R = root_wid.shape[0]
    x_root = _sc_gather(embedding, root_wid)
    hrnei = _sc_gather(h, root_mess_nei.reshape(-1)).reshape(-1, NB, H)
    root_vecs = _root_agg(R, x_root, hrnei, W_w, W_b)
    return (h, root_vecs)
